# baseline (speedup 1.0000x reference)
import jax
import jax.numpy as jnp
from jax.experimental import pallas as pl
from jax.experimental.pallas import tpu as pltpu

_B, _N, _C, _H, _W = 8, 64, 8, 64, 64
_M = 32
_HW = _H * _W
_NC = _C - 1
_NA = _N + 8
_MA = _M + 8


def _ot_kernel(gt_ref, s1_ref, s2_ref, s4_ref, s6_ref, prob_ref, pg_ref,
               out_ref, g2_ref, sa_ref, acc_ref):
    b = pl.program_id(0)

    @pl.when(b == 0)
    def _():
        r8 = jax.lax.broadcasted_iota(jnp.int32, (8, _HW), 0)
        sa_ref[_N:_NA, :] = jnp.where(r8 == 0, 1.0, 0.0).astype(jnp.bfloat16)
        g2_ref[_M + 1:_MA, :] = jnp.full((7, _HW), -1, jnp.int32)

    g2_ref[0:_M, :] = gt_ref[0].reshape(_M, _HW)

    def channel(c, s_in, ki):
        g2_ref[_M:_M + 1, :] = jnp.full((1, _HW), c, jnp.int32)
        sa_ref[0:_N, :] = (
            s_in[0, :, ki].astype(jnp.bfloat16).reshape(_N, _HW)
        )
        mask = (g2_ref[...] == c).astype(jnp.bfloat16)
        out = jax.lax.dot_general(
            sa_ref[...], mask,
            dimension_numbers=(((1,), (1,)), ((), ())),
            preferred_element_type=jnp.float32,
        )
        inter = out[0:_N, 0:_M]
        s_sum = out[0:_N, _M:_M + 1]
        g_sum = out[_N:_N + 1, 0:_M]
        union = s_sum + g_sum - inter
        return (inter + 1.0) / (union + 1.0)

    ratio = channel(1, s1_ref, 0)
    ratio += channel(2, s2_ref, 0)
    ratio += channel(3, s2_ref, 1)
    ratio += channel(4, s4_ref, 0)
    ratio += channel(5, s4_ref, 1)
    ratio += channel(6, s6_ref, 0)
    ratio += channel(7, s6_ref, 1)

    cost = 1.0 - ratio * (1.0 / _NC)
    minv = jnp.min(cost, axis=0, keepdims=True)
    iota_n = jax.lax.broadcasted_iota(jnp.int32, (_N, _M), 0)
    cand = jnp.where(cost <= minv, iota_n, _N)
    idx = jnp.min(cand, axis=0, keepdims=True)
    onehot = (iota_n == idx).astype(jnp.float32)

    pg = pg_ref[pl.ds(b, 1), :]
    seg_b = jnp.sum(minv * pg)

    target = jnp.sum(onehot * pg, axis=1, keepdims=True)
    p = prob_ref[pl.ds(b, 1), :].reshape(_N, 1)
    safe_t = jnp.where(target > 0, target, 1.0)
    kl_elem = jnp.where(
        target > 0, target * (jnp.log(safe_t) - jnp.log(p + 1e-8)), 0.0
    )
    kl_b = jnp.sum(kl_elem)

    @pl.when(b == 0)
    def _():
        acc_ref[0] = seg_b
        acc_ref[1] = kl_b

    @pl.when(b != 0)
    def _():
        acc_ref[0] += seg_b
        acc_ref[1] += kl_b

    @pl.when(b == _B - 1)
    def _():
        seg_loss = acc_ref[0] * (1.0 / _B)
        kl_loss = acc_ref[1] * (1.0 / (_B * _N))
        out_ref[0] = seg_loss + kl_loss
        out_ref[1] = seg_loss
        out_ref[2] = kl_loss


def kernel(gt_arr, sample_arr, prob, prob_gt, sample_shape):
    del sample_shape
    out = pl.pallas_call(
        _ot_kernel,
        grid=(_B,),
        in_specs=[
            pl.BlockSpec((1, _M, _H, _W), lambda b: (b, 0, 0, 0)),
            pl.BlockSpec((1, _N, 1, _H, _W), lambda b: (b, 0, 1, 0, 0)),
            pl.BlockSpec((1, _N, 2, _H, _W), lambda b: (b, 0, 1, 0, 0)),
            pl.BlockSpec((1, _N, 2, _H, _W), lambda b: (b, 0, 2, 0, 0)),
            pl.BlockSpec((1, _N, 2, _H, _W), lambda b: (b, 0, 3, 0, 0)),
            pl.BlockSpec((_B, _N), lambda b: (0, 0)),
            pl.BlockSpec((_B, _M), lambda b: (0, 0)),
        ],
        out_specs=pl.BlockSpec(memory_space=pltpu.SMEM),
        out_shape=jax.ShapeDtypeStruct((3,), jnp.float32),
        scratch_shapes=[
            pltpu.VMEM((_MA, _HW), jnp.int32),
            pltpu.VMEM((_NA, _HW), jnp.bfloat16),
            pltpu.SMEM((2,), jnp.float32),
        ],
    )(gt_arr, sample_arr, sample_arr, sample_arr, sample_arr, prob, prob_gt)
    return (out[0], out[1], out[2])
